# Initial kernel scaffold; baseline (speedup 1.0000x reference)
import jax
import jax.numpy as jnp
from jax.experimental import pallas as pl


def kernel(embeds):
    raise NotImplementedError("write your pallas kernel here")


# speedup vs baseline: 1.2050x; 1.2050x over previous
import functools
from itertools import product

import jax
import jax.numpy as jnp
from jax import lax
from jax.experimental import pallas as pl
from jax.experimental.pallas import tpu as pltpu
from jax.experimental.pallas import tpu_sc as plsc

_E = 100
_N = 16384
_D = 32
_NW = 32
_SPW = _N // _NW
_NVEC = _SPW // 16
_NJ = 28
_NSLOT = 4

_mesh = plsc.VectorSubcoreMesh(core_axis_name="c", subcore_axis_name="s")


def _offsets(j):
    jc = jnp.where(j >= 27, 13, j)
    dr = jc // 9 - 1
    dg = (jc // 3) % 3 - 1
    db = jc % 3 - 1
    return dr, dg, db


@functools.partial(
    pl.kernel,
    mesh=_mesh,
    out_type=jax.ShapeDtypeStruct((_NW, 16), jnp.float32),
    compiler_params=pltpu.CompilerParams(use_tc_tiling_on_sc=False),
    scratch_types=[
        pltpu.VMEM((_SPW,), jnp.int32),
        pltpu.VMEM((_SPW,), jnp.int32),
        pltpu.VMEM((_SPW,), jnp.int32),
        pltpu.VMEM((_SPW,), jnp.int32),
        pltpu.VMEM((_SPW, _D), jnp.float32),
        pltpu.VMEM((_SPW,), jnp.int32),
        pltpu.VMEM((_SPW,), jnp.int32),
        pltpu.VMEM((_SPW,), jnp.int32),
        pltpu.VMEM((_SPW,), jnp.int32),
        pltpu.VMEM((_SPW, _D), jnp.float32),
        pltpu.VMEM((_SPW, _D), jnp.float32),
        pltpu.VMEM((_SPW, _D), jnp.float32),
        pltpu.VMEM((_SPW, _D), jnp.float32),
        pltpu.VMEM((16,), jnp.float32),
        pltpu.SemaphoreType.DMA,
        pltpu.SemaphoreType.DMA,
        pltpu.SemaphoreType.DMA,
        pltpu.SemaphoreType.DMA,
        pltpu.SemaphoreType.DMA,
    ],
)
def _sc_loss(embeds, r_hbm, g_hbm, b_hbm, out,
             r_v, g_v, b_v, ci, crow,
             i0, i1, i2, i3, b0, b1, b2, b3,
             part, semc, s0, s1, s2, s3):
    idx_bufs = (i0, i1, i2, i3)
    row_bufs = (b0, b1, b2, b3)
    sems = (s0, s1, s2, s3)

    wid = lax.axis_index("s") * 2 + lax.axis_index("c")
    base = wid * _SPW
    pltpu.sync_copy(r_hbm.at[pl.ds(base, _SPW)], r_v)
    pltpu.sync_copy(g_hbm.at[pl.ds(base, _SPW)], g_v)
    pltpu.sync_copy(b_hbm.at[pl.ds(base, _SPW)], b_v)

    def fill_idx(j, dst):
        dr, dg, db = _offsets(j)

        def body(i, carry):
            sl = pl.ds(i * 16, 16)
            rr = jnp.clip(r_v[sl] + dr, 0, _E - 1)
            gg = jnp.clip(g_v[sl] + dg, 0, _E - 1)
            bb = jnp.clip(b_v[sl] + db, 0, _E - 1)
            dst[sl] = rr + gg * _E + bb * (_E * _E)
            return carry

        lax.fori_loop(0, _NVEC, body, 0)

    def accum(acc, rows):
        def body(s, a):
            for h in range(2):
                sl = pl.ds(h * 16, 16)
                d = crow[s, sl] - rows[s, sl]
                a = a + d * d
            return a

        return lax.fori_loop(0, _SPW, body, acc)

    fill_idx(13, ci)
    ccopy = pltpu.async_copy(embeds.at[ci], crow, semc)
    for jj in range(_NSLOT):
        fill_idx(jj, idx_bufs[jj])
        pltpu.async_copy(embeds.at[idx_bufs[jj]], row_bufs[jj], sems[jj])
    ccopy.wait()

    def outer(t, acc):
        for jj in range(_NSLOT):
            j = t * _NSLOT + jj
            pltpu.make_async_copy(
                embeds.at[idx_bufs[jj]], row_bufs[jj], sems[jj]).wait()
            acc = accum(acc, row_bufs[jj])
            fill_idx(j + _NSLOT, idx_bufs[jj])
            pltpu.async_copy(embeds.at[idx_bufs[jj]], row_bufs[jj], sems[jj])
        return acc

    acc = lax.fori_loop(0, _NJ // _NSLOT - 1, outer,
                        jnp.zeros((16,), jnp.float32))
    for jj in range(_NSLOT):
        pltpu.make_async_copy(
            embeds.at[idx_bufs[jj]], row_bufs[jj], sems[jj]).wait()
        acc = accum(acc, row_bufs[jj])

    part[...] = acc
    pltpu.sync_copy(part, out.at[wid])


def kernel(embeds):
    k_rgb = jax.random.fold_in(jax.random.key(0), 1)
    rgb = jax.random.randint(k_rgb, (_N, 3), 0, _E, dtype=jnp.int32)
    parts = _sc_loss(embeds, rgb[:, 0], rgb[:, 1], rgb[:, 2])
    return jnp.sqrt(jnp.sum(parts))


# speedup vs baseline: 1.2286x; 1.0196x over previous
import functools
from itertools import product

import jax
import jax.numpy as jnp
from jax import lax
from jax.experimental import pallas as pl
from jax.experimental.pallas import tpu as pltpu
from jax.experimental.pallas import tpu_sc as plsc

_E = 100
_N = 16384
_D = 32
_NW = 32
_SPW = _N // _NW
_NVEC = _SPW // 16
_NJ = 28
_NSLOT = 4

_mesh = plsc.VectorSubcoreMesh(core_axis_name="c", subcore_axis_name="s")


def _offsets(j):
    jc = jnp.where(j >= 27, 13, j)
    dr = jc // 9 - 1
    dg = (jc // 3) % 3 - 1
    db = jc % 3 - 1
    return dr, dg, db


@functools.partial(
    pl.kernel,
    mesh=_mesh,
    out_type=jax.ShapeDtypeStruct((_NW, 16), jnp.float32),
    compiler_params=pltpu.CompilerParams(use_tc_tiling_on_sc=False),
    scratch_types=[
        pltpu.VMEM((_SPW,), jnp.int32),
        pltpu.VMEM((_SPW,), jnp.int32),
        pltpu.VMEM((_SPW,), jnp.int32),
        pltpu.VMEM((_SPW,), jnp.int32),
        pltpu.VMEM((_SPW, _D), jnp.float32),
        pltpu.VMEM((_SPW,), jnp.int32),
        pltpu.VMEM((_SPW,), jnp.int32),
        pltpu.VMEM((_SPW,), jnp.int32),
        pltpu.VMEM((_SPW,), jnp.int32),
        pltpu.VMEM((_SPW, _D), jnp.float32),
        pltpu.VMEM((_SPW, _D), jnp.float32),
        pltpu.VMEM((_SPW, _D), jnp.float32),
        pltpu.VMEM((_SPW, _D), jnp.float32),
        pltpu.VMEM((16,), jnp.float32),
        pltpu.SemaphoreType.DMA,
        pltpu.SemaphoreType.DMA,
        pltpu.SemaphoreType.DMA,
        pltpu.SemaphoreType.DMA,
        pltpu.SemaphoreType.DMA,
    ],
)
def _sc_loss(embeds, r_hbm, g_hbm, b_hbm, out,
             r_v, g_v, b_v, ci, crow,
             i0, i1, i2, i3, b0, b1, b2, b3,
             part, semc, s0, s1, s2, s3):
    idx_bufs = (i0, i1, i2, i3)
    row_bufs = (b0, b1, b2, b3)
    sems = (s0, s1, s2, s3)

    wid = lax.axis_index("s") * 2 + lax.axis_index("c")
    base = wid * _SPW
    pltpu.sync_copy(r_hbm.at[pl.ds(base, _SPW)], r_v)
    pltpu.sync_copy(g_hbm.at[pl.ds(base, _SPW)], g_v)
    pltpu.sync_copy(b_hbm.at[pl.ds(base, _SPW)], b_v)

    def fill_idx(j, dst):
        dr, dg, db = _offsets(j)

        def body(i, carry):
            sl = pl.ds(i * 16, 16)
            rr = jnp.clip(r_v[sl] + dr, 0, _E - 1)
            gg = jnp.clip(g_v[sl] + dg, 0, _E - 1)
            bb = jnp.clip(b_v[sl] + db, 0, _E - 1)
            dst[sl] = rr + gg * _E + bb * (_E * _E)
            return carry

        lax.fori_loop(0, _NVEC, body, 0, unroll=8)

    def accum(acc, rows):
        def body(s, a):
            for h in range(2):
                sl = pl.ds(h * 16, 16)
                d = crow[s, sl] - rows[s, sl]
                a = a + d * d
            return a

        return lax.fori_loop(0, _SPW, body, acc, unroll=8)

    fill_idx(13, ci)
    ccopy = pltpu.async_copy(embeds.at[ci], crow, semc)
    for jj in range(_NSLOT):
        fill_idx(jj, idx_bufs[jj])
        pltpu.async_copy(embeds.at[idx_bufs[jj]], row_bufs[jj], sems[jj])
    ccopy.wait()

    def outer(t, acc):
        for jj in range(_NSLOT):
            j = t * _NSLOT + jj
            pltpu.make_async_copy(
                embeds.at[idx_bufs[jj]], row_bufs[jj], sems[jj]).wait()
            acc = accum(acc, row_bufs[jj])
            fill_idx(j + _NSLOT, idx_bufs[jj])
            pltpu.async_copy(embeds.at[idx_bufs[jj]], row_bufs[jj], sems[jj])
        return acc

    acc = lax.fori_loop(0, _NJ // _NSLOT - 1, outer,
                        jnp.zeros((16,), jnp.float32))
    for jj in range(_NSLOT):
        pltpu.make_async_copy(
            embeds.at[idx_bufs[jj]], row_bufs[jj], sems[jj]).wait()
        acc = accum(acc, row_bufs[jj])

    part[...] = acc
    pltpu.sync_copy(part, out.at[wid])


def kernel(embeds):
    k_rgb = jax.random.fold_in(jax.random.key(0), 1)
    rgb = jax.random.randint(k_rgb, (_N, 3), 0, _E, dtype=jnp.int32)
    parts = _sc_loss(embeds, rgb[:, 0], rgb[:, 1], rgb[:, 2])
    return jnp.sqrt(jnp.sum(parts))
